# baseline (speedup 1.0000x reference)
import dataclasses
import functools

import jax
import jax.numpy as jnp
from jax import lax
from jax.experimental import pallas as pl
from jax.experimental.pallas import tpu as pltpu
from jax.experimental.pallas import tpu_sc as plsc

B = 16384
K = 64
L = 16
NC = 2
NS = 16
NW = NC * NS
BPW = B // NW
CHUNK = 128
NCH = BPW // CHUNK


def _mf_score_body(u_hbm, i_hbm, U_hbm, I_hbm, out_hbm,
                   uidx_v, iidx_v, urow_v, irow_v, part_v, out_v, sem):
    wid = lax.axis_index("s") * NC + lax.axis_index("c")
    base = wid * BPW
    for c in range(NCH):
        pltpu.sync_copy(u_hbm.at[pl.ds(base + c * CHUNK, CHUNK)], uidx_v)
        pltpu.sync_copy(i_hbm.at[pl.ds(base + c * CHUNK, CHUNK)], iidx_v)
        cp_u = pltpu.async_copy(U_hbm.at[uidx_v], urow_v, sem)
        cp_i = pltpu.async_copy(I_hbm.at[iidx_v], irow_v, sem)
        cp_u.wait()
        cp_i.wait()

        @pl.loop(0, CHUNK)
        def _(r):
            s = urow_v[r, pl.ds(0, L)] * irow_v[r, pl.ds(0, L)]
            for k in range(1, K // L):
                s = s + urow_v[r, pl.ds(k * L, L)] * irow_v[r, pl.ds(k * L, L)]
            part_v[r, :] = s

        @pl.loop(0, CHUNK // L)
        def _(g):
            rows = lax.iota(jnp.int32, L) + g * L
            acc = plsc.load_gather(part_v, [rows, jnp.zeros((L,), jnp.int32)])
            for k in range(1, L):
                acc = acc + plsc.load_gather(
                    part_v, [rows, jnp.full((L,), k, jnp.int32)])
            out_v[pl.ds(c * CHUNK + g * L, L)] = acc

    pltpu.sync_copy(out_v, out_hbm.at[pl.ds(base, BPW)])


@jax.jit
def _mf_score(u, i, U, I):
    mesh = plsc.VectorSubcoreMesh(core_axis_name="c", subcore_axis_name="s")
    cp = pltpu.CompilerParams(
        needs_layout_passes=False,
        use_tc_tiling_on_sc=False,
    )
    run = pl.kernel(
        _mf_score_body,
        out_type=jax.ShapeDtypeStruct((B,), jnp.float32),
        mesh=mesh,
        scratch_types=[
            pltpu.VMEM((CHUNK,), jnp.int32),
            pltpu.VMEM((CHUNK,), jnp.int32),
            pltpu.VMEM((CHUNK, K), jnp.float32),
            pltpu.VMEM((CHUNK, K), jnp.float32),
            pltpu.VMEM((CHUNK, L), jnp.float32),
            pltpu.VMEM((BPW,), jnp.float32),
            pltpu.SemaphoreType.DMA,
        ],
        compiler_params=cp,
    )
    return run(u, i, U, I)


def kernel(u, i, U, I):
    return _mf_score(u, i, U, I)
